# Initial kernel scaffold; baseline (speedup 1.0000x reference)
import functools

import jax
import jax.numpy as jnp
from jax import lax
from jax.experimental import pallas as pl
from jax.experimental.pallas import tpu as pltpu
from jax.experimental.pallas import tpu_sc as plsc

N = 10000
E = 320000
D = 128
Q = 32
NQ = D // Q
NC = 2
NS = 16
L = 16
B = 512
JR = B // 128
NCHUNK = E // B
CPS = (NCHUNK + NS - 1) // NS
RPS = N // NS


def _sc_body(node_hbm, edge_hbm, src_hbm, dst_hbm, ho_hbm, hi_hbm,
             acc_hon, acc_hoe, acc_hin, acc_hie,
             idx_s, idx_d, idx_gs, idx_gd, ns, nd, ee, v1, v2):
    c = lax.axis_index("c")
    s = lax.axis_index("s")
    row0 = s * RPS

    for p in range(2):
        q = 2 * c + p

        @pl.loop(0, RPS)
        def _zero_rows(r):
            @pl.loop(0, Q, step=L)
            def _zero_cols(j):
                v1[r, pl.ds(j, L)] = jnp.zeros((L,), jnp.float32)

        pltpu.sync_copy(v1, acc_hon.at[pl.ds(row0, RPS)])
        pltpu.sync_copy(v1, acc_hoe.at[pl.ds(row0, RPS)])
        pltpu.sync_copy(v1, acc_hin.at[pl.ds(row0, RPS)])
        pltpu.sync_copy(v1, acc_hie.at[pl.ds(row0, RPS)])
        plsc.subcore_barrier()

        @pl.loop(0, CPS)
        def _chunk(i):
            ci = s + i * NS

            @pl.when(ci < NCHUNK)
            def _():
                base = ci * B
                irow = ci * JR
                pltpu.sync_copy(src_hbm.at[pl.ds(irow, JR)], idx_s)
                pltpu.sync_copy(dst_hbm.at[pl.ds(irow, JR)], idx_d)

                @pl.loop(0, JR)
                def _gidx(j):
                    @pl.loop(0, 128, step=L)
                    def _gidx2(k):
                        idx_gs[j, pl.ds(k, L)] = idx_s[j, pl.ds(k, L)] * NQ + q
                        idx_gd[j, pl.ds(k, L)] = idx_d[j, pl.ds(k, L)] * NQ + q

                pltpu.sync_copy(edge_hbm.at[pl.ds(base, B), q], ee)

                for j in range(JR):
                    pltpu.sync_copy(node_hbm.at[idx_gs.at[j]],
                                    ns.at[pl.ds(j * 128, 128)])
                    pltpu.sync_copy(node_hbm.at[idx_gd.at[j]],
                                    nd.at[pl.ds(j * 128, 128)])
                for j in range(JR):
                    blk = pl.ds(j * 128, 128)
                    pltpu.sync_copy(ns.at[blk], acc_hon.at[idx_d.at[j]], add=True)
                    pltpu.sync_copy(nd.at[blk], acc_hin.at[idx_s.at[j]], add=True)
                    pltpu.sync_copy(ee.at[blk], acc_hoe.at[idx_d.at[j]], add=True)
                    pltpu.sync_copy(ee.at[blk], acc_hie.at[idx_s.at[j]], add=True)

        plsc.subcore_barrier()

        for accn, acce, out in ((acc_hon, acc_hoe, ho_hbm),
                                (acc_hin, acc_hie, hi_hbm)):
            pltpu.sync_copy(accn.at[pl.ds(row0, RPS)], v1)
            pltpu.sync_copy(acce.at[pl.ds(row0, RPS)], v2)

            @pl.loop(0, RPS)
            def _comb(r):
                @pl.loop(0, Q, step=L)
                def _comb2(j):
                    v1[r, pl.ds(j, L)] = v1[r, pl.ds(j, L)] - v2[r, pl.ds(j, L)]

            pltpu.sync_copy(v1, out.at[pl.ds(row0, RPS), pl.ds(q * Q, Q)])
        plsc.subcore_barrier()


@jax.jit
def _sc_segments(node_flat, edge4, src2, dst2):
    mesh = plsc.VectorSubcoreMesh(core_axis_name="c", subcore_axis_name="s",
                                  num_cores=NC, num_subcores=NS)
    f32 = jnp.float32
    run = pl.kernel(
        _sc_body,
        out_type=(jax.ShapeDtypeStruct((N, D), f32),
                  jax.ShapeDtypeStruct((N, D), f32)),
        mesh=mesh,
        scratch_types=[
            pltpu.VMEM_SHARED((N, Q), f32),
            pltpu.VMEM_SHARED((N, Q), f32),
            pltpu.VMEM_SHARED((N, Q), f32),
            pltpu.VMEM_SHARED((N, Q), f32),
            pltpu.VMEM((JR, 128), jnp.int32),
            pltpu.VMEM((JR, 128), jnp.int32),
            pltpu.VMEM((JR, 128), jnp.int32),
            pltpu.VMEM((JR, 128), jnp.int32),
            pltpu.VMEM((B, Q), f32),
            pltpu.VMEM((B, Q), f32),
            pltpu.VMEM((B, Q), f32),
            pltpu.VMEM((RPS, Q), f32),
            pltpu.VMEM((RPS, Q), f32),
        ],
    )
    return run(node_flat, edge4, src2, dst2)


def _he_body(x_ref, w_ref, b_ref, o_ref):
    o_ref[...] = lax.dot_general(
        x_ref[...], w_ref[...], (((1,), (1,)), ((), ())),
        preferred_element_type=jnp.float32) + b_ref[...]


def _h_body(ho_ref, hi_ref, wo_ref, wi_ref, b_ref, o_ref):
    o_ref[...] = (
        lax.dot_general(ho_ref[...], wo_ref[...], (((1,), (1,)), ((), ())),
                        preferred_element_type=jnp.float32)
        + lax.dot_general(hi_ref[...], wi_ref[...], (((1,), (1,)), ((), ())),
                          preferred_element_type=jnp.float32)
        + b_ref[...])


BE = 4000
BN = 2000


@jax.jit
def _tc_he(edge_embs, W_rel, b_rel):
    return pl.pallas_call(
        _he_body,
        grid=(E // BE,),
        in_specs=[
            pl.BlockSpec((BE, D), lambda i: (i, 0)),
            pl.BlockSpec((D, D), lambda i: (0, 0)),
            pl.BlockSpec((1, D), lambda i: (0, 0)),
        ],
        out_specs=pl.BlockSpec((BE, D), lambda i: (i, 0)),
        out_shape=jax.ShapeDtypeStruct((E, D), jnp.float32),
    )(edge_embs, W_rel, b_rel.reshape(1, D))


@jax.jit
def _tc_h(ho, hi, W_O, W_I, b):
    return pl.pallas_call(
        _h_body,
        grid=(N // BN,),
        in_specs=[
            pl.BlockSpec((BN, D), lambda i: (i, 0)),
            pl.BlockSpec((BN, D), lambda i: (i, 0)),
            pl.BlockSpec((D, D), lambda i: (0, 0)),
            pl.BlockSpec((D, D), lambda i: (0, 0)),
            pl.BlockSpec((1, D), lambda i: (0, 0)),
        ],
        out_specs=pl.BlockSpec((BN, D), lambda i: (i, 0)),
        out_shape=jax.ShapeDtypeStruct((N, D), jnp.float32),
    )(ho, hi, W_O, W_I, b.reshape(1, D))


def kernel(node_embs, edge_index, edge_embs, W_O, b_O, W_I, b_I, W_rel, b_rel):
    node_flat = node_embs.reshape(N * NQ, Q)
    edge4 = edge_embs.reshape(E, NQ, Q)
    src2 = edge_index[0].reshape(E // 128, 128)
    dst2 = edge_index[1].reshape(E // 128, 128)
    ho, hi = _sc_segments(node_flat, edge4, src2, dst2)
    h = _tc_h(ho, hi, W_O, W_I, b_O + b_I)
    he = _tc_he(edge_embs, W_rel, b_rel)
    return (h, he)


# speedup vs baseline: 2.5367x; 2.5367x over previous
import jax
import jax.numpy as jnp
from jax import lax
from jax.experimental import pallas as pl
from jax.experimental.pallas import tpu as pltpu
from jax.experimental.pallas import tpu_sc as plsc

N = 10000
E = 320000
D = 128
Q = 32
NQ = D // Q
NC = 2
NS = 16
L = 16
B = 256
JR = B // 128
NCHUNK = E // B
M = 80
RA = 624
RB = 640


def _combine_dump(accn, acce, out, ns, nd, row0, obase, piece, npieces):
    for t in range(npieces):
        pltpu.sync_copy(accn.at[pl.ds(row0 + t * piece, piece)],
                        ns.at[pl.ds(0, piece)])
        pltpu.sync_copy(acce.at[pl.ds(row0 + t * piece, piece)],
                        nd.at[pl.ds(0, piece)])

        @pl.loop(0, piece)
        def _comb(r):
            @pl.loop(0, Q, step=L)
            def _comb2(k):
                ns[r, pl.ds(k, L)] = ns[r, pl.ds(k, L)] - nd[r, pl.ds(k, L)]

        pltpu.sync_copy(ns.at[pl.ds(0, piece)],
                        out.at[pl.ds(obase + t * piece, piece)])


def _zero_slice(accs, zbuf, row0, piece, npieces):
    for acc in accs:
        for t in range(npieces):
            pltpu.sync_copy(zbuf.at[pl.ds(0, piece)],
                            acc.at[pl.ds(row0 + t * piece, piece)])


def _sc_body(node_hbm, edge_hbm, src_hbm, dst_hbm, ho_hbm, hi_hbm,
             acc_hon, acc_hoe, acc_hin, acc_hie,
             idx_s, idx_d, idx_gs, idx_gd, idx_e, ramp,
             ns, nd, ee):
    c = lax.axis_index("c")
    s = lax.axis_index("s")
    row0 = pl.multiple_of(s * RA, 8)
    last = s == NS - 1
    accs = (acc_hon, acc_hoe, acc_hin, acc_hie)

    @pl.loop(0, JR)
    def _ramp_j(j):
        @pl.loop(0, 128, step=L)
        def _ramp_k(k):
            ramp[j, pl.ds(k, L)] = (jnp.arange(L, dtype=jnp.int32)
                                    + (j * 128 + k)) * NQ

    for p in range(2):
        q = 2 * c + p

        @pl.loop(0, B)
        def _zero_rows(r):
            @pl.loop(0, Q, step=L)
            def _zero_cols(k):
                ns[r, pl.ds(k, L)] = jnp.zeros((L,), jnp.float32)

        @pl.when(jnp.logical_not(last))
        def _():
            _zero_slice(accs, ns, row0, RA // 3, 3)

        @pl.when(last)
        def _():
            _zero_slice(accs, ns, row0, RB // 4, 4)
        plsc.subcore_barrier()

        @pl.loop(0, M)
        def _chunk(li):
            ci = s * M + li

            @pl.when(ci < NCHUNK)
            def _():
                ebase = ci * B * NQ + q
                irow = ci * JR
                pltpu.sync_copy(src_hbm.at[pl.ds(irow, JR)], idx_s)
                pltpu.sync_copy(dst_hbm.at[pl.ds(irow, JR)], idx_d)

                @pl.loop(0, JR)
                def _gidx(j):
                    @pl.loop(0, 128, step=L)
                    def _gidx2(k):
                        idx_gs[j, pl.ds(k, L)] = idx_s[j, pl.ds(k, L)] * NQ + q
                        idx_gd[j, pl.ds(k, L)] = idx_d[j, pl.ds(k, L)] * NQ + q
                        idx_e[j, pl.ds(k, L)] = ramp[j, pl.ds(k, L)] + ebase

                for j in range(JR):
                    blk = pl.ds(j * 128, 128)
                    pltpu.sync_copy(node_hbm.at[idx_gs.at[j]], ns.at[blk])
                    pltpu.sync_copy(node_hbm.at[idx_gd.at[j]], nd.at[blk])
                    pltpu.sync_copy(edge_hbm.at[idx_e.at[j]], ee.at[blk])
                for j in range(JR):
                    blk = pl.ds(j * 128, 128)
                    pltpu.sync_copy(ns.at[blk], acc_hon.at[idx_d.at[j]], add=True)
                    pltpu.sync_copy(nd.at[blk], acc_hin.at[idx_s.at[j]], add=True)
                    pltpu.sync_copy(ee.at[blk], acc_hoe.at[idx_d.at[j]], add=True)
                    pltpu.sync_copy(ee.at[blk], acc_hie.at[idx_s.at[j]], add=True)

        plsc.subcore_barrier()

        obase = pl.multiple_of(q * N + row0, 8)

        @pl.when(jnp.logical_not(last))
        def _():
            _combine_dump(acc_hon, acc_hoe, ho_hbm, ns, nd, row0, obase,
                          RA // 3, 3)
            _combine_dump(acc_hin, acc_hie, hi_hbm, ns, nd, row0, obase,
                          RA // 3, 3)

        @pl.when(last)
        def _():
            _combine_dump(acc_hon, acc_hoe, ho_hbm, ns, nd, row0, obase,
                          RB // 4, 4)
            _combine_dump(acc_hin, acc_hie, hi_hbm, ns, nd, row0, obase,
                          RB // 4, 4)
        plsc.subcore_barrier()


@jax.jit
def _sc_segments(node_flat, edge_flat, src2, dst2):
    mesh = plsc.VectorSubcoreMesh(core_axis_name="c", subcore_axis_name="s",
                                  num_cores=NC, num_subcores=NS)
    f32 = jnp.float32
    i32 = jnp.int32
    run = pl.kernel(
        _sc_body,
        out_type=(jax.ShapeDtypeStruct((NQ * N, Q), f32),
                  jax.ShapeDtypeStruct((NQ * N, Q), f32)),
        mesh=mesh,
        compiler_params=pltpu.CompilerParams(use_tc_tiling_on_sc=False),
        scratch_types=[
            pltpu.VMEM_SHARED((N, Q), f32),
            pltpu.VMEM_SHARED((N, Q), f32),
            pltpu.VMEM_SHARED((N, Q), f32),
            pltpu.VMEM_SHARED((N, Q), f32),
            pltpu.VMEM((JR, 128), i32),
            pltpu.VMEM((JR, 128), i32),
            pltpu.VMEM((JR, 128), i32),
            pltpu.VMEM((JR, 128), i32),
            pltpu.VMEM((JR, 128), i32),
            pltpu.VMEM((JR, 128), i32),
            pltpu.VMEM((B, Q), f32),
            pltpu.VMEM((B, Q), f32),
            pltpu.VMEM((B, Q), f32),
        ],
    )
    return run(node_flat, edge_flat, src2, dst2)


def _he_body(x_ref, w_ref, b_ref, o_ref):
    o_ref[...] = lax.dot_general(
        x_ref[...], w_ref[...], (((1,), (1,)), ((), ())),
        preferred_element_type=jnp.float32) + b_ref[...]


def _h_body(ho_ref, hi_ref, wo_ref, wi_ref, b_ref, o_ref):
    o_ref[...] = (
        lax.dot_general(ho_ref[...], wo_ref[...], (((1,), (1,)), ((), ())),
                        preferred_element_type=jnp.float32)
        + lax.dot_general(hi_ref[...], wi_ref[...], (((1,), (1,)), ((), ())),
                          preferred_element_type=jnp.float32)
        + b_ref[...])


BE = 4000
BN = 2000


@jax.jit
def _tc_he(edge_embs, W_rel, b_rel):
    return pl.pallas_call(
        _he_body,
        grid=(E // BE,),
        in_specs=[
            pl.BlockSpec((BE, D), lambda i: (i, 0)),
            pl.BlockSpec((D, D), lambda i: (0, 0)),
            pl.BlockSpec((1, D), lambda i: (0, 0)),
        ],
        out_specs=pl.BlockSpec((BE, D), lambda i: (i, 0)),
        out_shape=jax.ShapeDtypeStruct((E, D), jnp.float32),
    )(edge_embs, W_rel, b_rel.reshape(1, D))


@jax.jit
def _tc_h(ho4, hi4, W_O, W_I, b):
    ho = ho4.reshape(NQ, N, Q).transpose(1, 0, 2).reshape(N, D)
    hi = hi4.reshape(NQ, N, Q).transpose(1, 0, 2).reshape(N, D)
    return pl.pallas_call(
        _h_body,
        grid=(N // BN,),
        in_specs=[
            pl.BlockSpec((BN, D), lambda i: (i, 0)),
            pl.BlockSpec((BN, D), lambda i: (i, 0)),
            pl.BlockSpec((D, D), lambda i: (0, 0)),
            pl.BlockSpec((D, D), lambda i: (0, 0)),
            pl.BlockSpec((1, D), lambda i: (0, 0)),
        ],
        out_specs=pl.BlockSpec((BN, D), lambda i: (i, 0)),
        out_shape=jax.ShapeDtypeStruct((N, D), jnp.float32),
    )(ho, hi, W_O, W_I, b.reshape(1, D))


def kernel(node_embs, edge_index, edge_embs, W_O, b_O, W_I, b_I, W_rel, b_rel):
    node_flat = node_embs.reshape(N * NQ, Q)
    edge_flat = edge_embs.reshape(E * NQ, Q)
    src2 = edge_index[0].reshape(E // 128, 128)
    dst2 = edge_index[1].reshape(E // 128, 128)
    ho4, hi4 = _sc_segments(node_flat, edge_flat, src2, dst2)
    h = _tc_h(ho4, hi4, W_O, W_I, b_O + b_I)
    he = _tc_he(edge_embs, W_rel, b_rel)
    return (h, he)


# speedup vs baseline: 4.7336x; 1.8661x over previous
import jax
import jax.numpy as jnp
from jax import lax
from jax.experimental import pallas as pl
from jax.experimental.pallas import tpu as pltpu
from jax.experimental.pallas import tpu_sc as plsc

N = 10000
E = 320000
D = 128
Q = 32
NQ = D // Q
NC = 2
NS = 16
L = 16
B = 256
JR = B // 128
NCHUNK = E // B
M = 80
RA = 624
RB = 640


def _combine_dump(accn, acce, out, ns, nd, row0, obase, piece, npieces):
    for t in range(npieces):
        pltpu.sync_copy(accn.at[pl.ds(row0 + t * piece, piece)],
                        ns.at[pl.ds(0, piece)])
        pltpu.sync_copy(acce.at[pl.ds(row0 + t * piece, piece)],
                        nd.at[pl.ds(0, piece)])

        @pl.loop(0, piece)
        def _comb(r):
            @pl.loop(0, Q, step=L)
            def _comb2(k):
                ns[r, pl.ds(k, L)] = ns[r, pl.ds(k, L)] - nd[r, pl.ds(k, L)]

        pltpu.sync_copy(ns.at[pl.ds(0, piece)],
                        out.at[pl.ds(obase + t * piece, piece)])


def _zero_slice(accs, zbuf, row0, piece, npieces):
    for acc in accs:
        for t in range(npieces):
            pltpu.sync_copy(zbuf.at[pl.ds(0, piece)],
                            acc.at[pl.ds(row0 + t * piece, piece)])


def _sc_body(node_hbm, edge_hbm, src_hbm, dst_hbm, ho_hbm, hi_hbm,
             acc_hon, acc_hoe, acc_hin, acc_hie,
             idx_s, idx_d, idx_gs, idx_gd, idx_e, ramp,
             ns, nd, ee, sem_i, sem_g, sem_s):
    c = lax.axis_index("c")
    s = lax.axis_index("s")
    row0 = pl.multiple_of(s * RA, 8)
    last = s == NS - 1
    accs = (acc_hon, acc_hoe, acc_hin, acc_hie)

    @pl.loop(0, JR)
    def _ramp_j(j):
        @pl.loop(0, 128, step=L)
        def _ramp_k(k):
            ramp[j, pl.ds(k, L)] = (jnp.arange(L, dtype=jnp.int32)
                                    + (j * 128 + k)) * NQ

    for p in range(2):
        q = 2 * c + p

        @pl.loop(0, B)
        def _zero_rows(r):
            @pl.loop(0, Q, step=L)
            def _zero_cols(k):
                ns[r, pl.ds(k, L)] = jnp.zeros((L,), jnp.float32)

        @pl.when(jnp.logical_not(last))
        def _():
            _zero_slice(accs, ns, row0, RA // 3, 3)

        @pl.when(last)
        def _():
            _zero_slice(accs, ns, row0, RB // 4, 4)
        plsc.subcore_barrier()

        @pl.loop(0, M)
        def _chunk(li):
            ci = s * M + li

            @pl.when(ci < NCHUNK)
            def _():
                ebase = ci * B * NQ + q
                irow = ci * JR
                di1 = pltpu.async_copy(src_hbm.at[pl.ds(irow, JR)], idx_s, sem_i)
                di2 = pltpu.async_copy(dst_hbm.at[pl.ds(irow, JR)], idx_d, sem_i)
                di1.wait()
                di2.wait()

                @pl.loop(0, JR)
                def _gidx(j):
                    @pl.loop(0, 128, step=L)
                    def _gidx2(k):
                        idx_gs[j, pl.ds(k, L)] = idx_s[j, pl.ds(k, L)] * NQ + q
                        idx_gd[j, pl.ds(k, L)] = idx_d[j, pl.ds(k, L)] * NQ + q
                        idx_e[j, pl.ds(k, L)] = ramp[j, pl.ds(k, L)] + ebase

                gathers = []
                for j in range(JR):
                    blk = pl.ds(j * 128, 128)
                    gathers.append(pltpu.async_copy(
                        node_hbm.at[idx_gs.at[j]], ns.at[blk], sem_g))
                    gathers.append(pltpu.async_copy(
                        node_hbm.at[idx_gd.at[j]], nd.at[blk], sem_g))
                    gathers.append(pltpu.async_copy(
                        edge_hbm.at[idx_e.at[j]], ee.at[blk], sem_g))
                for g in gathers:
                    g.wait()
                scatters = []
                for j in range(JR):
                    blk = pl.ds(j * 128, 128)
                    scatters.append(pltpu.async_copy(
                        ns.at[blk], acc_hon.at[idx_d.at[j]], sem_s, add=True))
                    scatters.append(pltpu.async_copy(
                        nd.at[blk], acc_hin.at[idx_s.at[j]], sem_s, add=True))
                    scatters.append(pltpu.async_copy(
                        ee.at[blk], acc_hoe.at[idx_d.at[j]], sem_s, add=True))
                    scatters.append(pltpu.async_copy(
                        ee.at[blk], acc_hie.at[idx_s.at[j]], sem_s, add=True))
                for sc in scatters:
                    sc.wait()

        plsc.subcore_barrier()

        obase = pl.multiple_of(q * N + row0, 8)

        @pl.when(jnp.logical_not(last))
        def _():
            _combine_dump(acc_hon, acc_hoe, ho_hbm, ns, nd, row0, obase,
                          RA // 3, 3)
            _combine_dump(acc_hin, acc_hie, hi_hbm, ns, nd, row0, obase,
                          RA // 3, 3)

        @pl.when(last)
        def _():
            _combine_dump(acc_hon, acc_hoe, ho_hbm, ns, nd, row0, obase,
                          RB // 4, 4)
            _combine_dump(acc_hin, acc_hie, hi_hbm, ns, nd, row0, obase,
                          RB // 4, 4)
        plsc.subcore_barrier()


@jax.jit
def _sc_segments(node_flat, edge_flat, src2, dst2):
    mesh = plsc.VectorSubcoreMesh(core_axis_name="c", subcore_axis_name="s",
                                  num_cores=NC, num_subcores=NS)
    f32 = jnp.float32
    i32 = jnp.int32
    run = pl.kernel(
        _sc_body,
        out_type=(jax.ShapeDtypeStruct((NQ * N, Q), f32),
                  jax.ShapeDtypeStruct((NQ * N, Q), f32)),
        mesh=mesh,
        compiler_params=pltpu.CompilerParams(use_tc_tiling_on_sc=False),
        scratch_types=[
            pltpu.VMEM_SHARED((N, Q), f32),
            pltpu.VMEM_SHARED((N, Q), f32),
            pltpu.VMEM_SHARED((N, Q), f32),
            pltpu.VMEM_SHARED((N, Q), f32),
            pltpu.VMEM((JR, 128), i32),
            pltpu.VMEM((JR, 128), i32),
            pltpu.VMEM((JR, 128), i32),
            pltpu.VMEM((JR, 128), i32),
            pltpu.VMEM((JR, 128), i32),
            pltpu.VMEM((JR, 128), i32),
            pltpu.VMEM((B, Q), f32),
            pltpu.VMEM((B, Q), f32),
            pltpu.VMEM((B, Q), f32),
            pltpu.SemaphoreType.DMA,
            pltpu.SemaphoreType.DMA,
            pltpu.SemaphoreType.DMA,
        ],
    )
    return run(node_flat, edge_flat, src2, dst2)


def _he_body(x_ref, w_ref, b_ref, o_ref):
    o_ref[...] = lax.dot_general(
        x_ref[...], w_ref[...], (((1,), (1,)), ((), ())),
        preferred_element_type=jnp.float32) + b_ref[...]


def _h_body(ho_ref, hi_ref, wo_ref, wi_ref, b_ref, o_ref):
    o_ref[...] = (
        lax.dot_general(ho_ref[...], wo_ref[...], (((1,), (1,)), ((), ())),
                        preferred_element_type=jnp.float32)
        + lax.dot_general(hi_ref[...], wi_ref[...], (((1,), (1,)), ((), ())),
                          preferred_element_type=jnp.float32)
        + b_ref[...])


BE = 4000
BN = 2000


@jax.jit
def _tc_he(edge_embs, W_rel, b_rel):
    return pl.pallas_call(
        _he_body,
        grid=(E // BE,),
        in_specs=[
            pl.BlockSpec((BE, D), lambda i: (i, 0)),
            pl.BlockSpec((D, D), lambda i: (0, 0)),
            pl.BlockSpec((1, D), lambda i: (0, 0)),
        ],
        out_specs=pl.BlockSpec((BE, D), lambda i: (i, 0)),
        out_shape=jax.ShapeDtypeStruct((E, D), jnp.float32),
    )(edge_embs, W_rel, b_rel.reshape(1, D))


@jax.jit
def _tc_h(ho4, hi4, W_O, W_I, b):
    ho = ho4.reshape(NQ, N, Q).transpose(1, 0, 2).reshape(N, D)
    hi = hi4.reshape(NQ, N, Q).transpose(1, 0, 2).reshape(N, D)
    return pl.pallas_call(
        _h_body,
        grid=(N // BN,),
        in_specs=[
            pl.BlockSpec((BN, D), lambda i: (i, 0)),
            pl.BlockSpec((BN, D), lambda i: (i, 0)),
            pl.BlockSpec((D, D), lambda i: (0, 0)),
            pl.BlockSpec((D, D), lambda i: (0, 0)),
            pl.BlockSpec((1, D), lambda i: (0, 0)),
        ],
        out_specs=pl.BlockSpec((BN, D), lambda i: (i, 0)),
        out_shape=jax.ShapeDtypeStruct((N, D), jnp.float32),
    )(ho, hi, W_O, W_I, b.reshape(1, D))


def kernel(node_embs, edge_index, edge_embs, W_O, b_O, W_I, b_I, W_rel, b_rel):
    node_flat = node_embs.reshape(N * NQ, Q)
    edge_flat = edge_embs.reshape(E * NQ, Q)
    src2 = edge_index[0].reshape(E // 128, 128)
    dst2 = edge_index[1].reshape(E // 128, 128)
    ho4, hi4 = _sc_segments(node_flat, edge_flat, src2, dst2)
    h = _tc_h(ho4, hi4, W_O, W_I, b_O + b_I)
    he = _tc_he(edge_embs, W_rel, b_rel)
    return (h, he)
